# baseline (speedup 1.0000x reference)
import functools

import jax
import jax.numpy as jnp
from jax import lax
from jax.experimental import pallas as pl
from jax.experimental.pallas import tpu as pltpu

_LANE = 128
_SUBLANE = 8
_TSPLIT = 12


def _ceil_to(x, m):
    return (x + m - 1) // m * m


def _head_kernel(tok_ref, emb_hbm, w_ih_ref, w_hh_ref, b_rnn_ref,
                 h_out_ref, emb_vmem, slab_a, slab_b, sem,
                 *, TS, Bt, Bp, dma_chunks):
    i = pl.program_id(0)
    base = i * Bt
    Vp = emb_vmem.shape[0]
    rows = Vp // dma_chunks
    copies = [
        pltpu.make_async_copy(emb_hbm.at[pl.ds(k * rows, rows), :],
                              emb_vmem.at[pl.ds(k * rows, rows), :],
                              sem.at[k])
        for k in range(dma_chunks)
    ]
    for c in copies:
        c.start()
    for c in copies:
        c.wait()

    def gather(slab, t):
        rowbase = t * Bp + base
        for b in range(Bt):
            idx = tok_ref[rowbase + b]
            slab[pl.ds(b, 1), :] = emb_vmem[pl.ds(idx, 1), :]

    def step(h, slab):
        pre = (jnp.dot(slab[...], w_ih_ref[...],
                       preferred_element_type=jnp.float32)
               + jnp.dot(h, w_hh_ref[...],
                         preferred_element_type=jnp.float32)
               + b_rnn_ref[...])
        return jnp.tanh(pre).astype(jnp.bfloat16)

    gather(slab_a, 0)
    h = jnp.zeros((Bt, w_hh_ref.shape[0]), jnp.bfloat16)
    for t in range(TS):
        slab_cur = slab_a if t % 2 == 0 else slab_b
        slab_nxt = slab_b if t % 2 == 0 else slab_a
        if t + 1 < TS:
            gather(slab_nxt, t + 1)
        h = step(h, slab_cur)
    h_out_ref[...] = h.astype(jnp.float32)


def _tail_kernel(x_ref, h_in_ref, w_ih_ref, w_hh_ref, b_rnn_ref,
                 w_fc_ref, b_fc_ref, out_ref, xw_ref, *, unroll):
    T, Bt, E = x_ref.shape
    xw = jnp.dot(x_ref[...].reshape(T * Bt, E), w_ih_ref[...],
                 preferred_element_type=jnp.float32)
    xw_ref[...] = (xw + b_rnn_ref[...]).astype(xw_ref.dtype).reshape(
        T, Bt, xw_ref.shape[2])

    def stepf(t, h):
        pre = xw_ref[t].astype(jnp.float32) + jnp.dot(
            h, w_hh_ref[...], preferred_element_type=jnp.float32)
        return jnp.tanh(pre).astype(h.dtype)

    h = lax.fori_loop(0, T, stepf, h_in_ref[...].astype(jnp.bfloat16),
                      unroll=unroll)
    out_ref[...] = (jnp.dot(h, w_fc_ref[...],
                            preferred_element_type=jnp.float32)
                    + b_fc_ref[...]).astype(out_ref.dtype)


def kernel(x_tokens, embedding, w_ih, w_hh, b_ih, b_hh, w_fc, b_fc):
    B, T = x_tokens.shape
    V, E = embedding.shape
    H = w_hh.shape[0]
    C = w_fc.shape[1]
    TS = min(_TSPLIT, T - 1)

    Ep, Hp, Cp = (_ceil_to(d, _LANE) for d in (E, H, C))
    Bt = min(256, _ceil_to(B, _SUBLANE))
    Bp = _ceil_to(B, Bt)
    num_tiles = Bp // Bt

    def padc(a, r, c):
        return jnp.pad(a, ((0, r - a.shape[0]), (0, c - a.shape[1])))

    tok_tm = x_tokens.T
    if Bp != B:
        tok_tm = jnp.pad(tok_tm, ((0, 0), (0, Bp - B)))
    tok_head = tok_tm[:TS].reshape(-1)
    emb_p = padc(embedding, _ceil_to(V, _SUBLANE), Ep)
    Vp = emb_p.shape[0]
    w_ih_c = padc(w_ih, Ep, Hp)
    w_hh_c = padc(w_hh, Hp, Hp).astype(jnp.bfloat16)
    w_fc_c = padc(w_fc, Hp, Cp).astype(jnp.bfloat16)
    b_rnn = padc(b_ih + b_hh, 1, Hp)
    b_fc_p = padc(b_fc, 1, Cp)

    x_tail = jnp.take(embedding, tok_tm[TS:], axis=0)
    if Ep != E:
        x_tail = jnp.pad(x_tail, ((0, 0), (0, 0), (0, Ep - E)))

    const2 = lambda i, *_: (0, 0)
    n_chunks = 8 if (Vp // 8) % _SUBLANE == 0 else 1
    head_spec = pltpu.PrefetchScalarGridSpec(
        num_scalar_prefetch=1,
        grid=(num_tiles,),
        in_specs=[
            pl.BlockSpec(memory_space=pl.ANY),
            pl.BlockSpec((Ep, Hp), const2),
            pl.BlockSpec((Hp, Hp), const2),
            pl.BlockSpec((1, Hp), const2),
        ],
        out_specs=pl.BlockSpec((Bt, Hp), lambda i, *_: (i, 0)),
        scratch_shapes=[
            pltpu.VMEM((Vp, Ep), jnp.float32),
            pltpu.VMEM((Bt, Ep), jnp.float32),
            pltpu.VMEM((Bt, Ep), jnp.float32),
            pltpu.SemaphoreType.DMA((8,)),
        ],
    )
    h_mid = pl.pallas_call(
        functools.partial(_head_kernel, TS=TS, Bt=Bt, Bp=Bp,
                          dma_chunks=n_chunks),
        out_shape=jax.ShapeDtypeStruct((Bp, Hp), jnp.float32),
        grid_spec=head_spec,
        compiler_params=pltpu.CompilerParams(
            dimension_semantics=("parallel",),
            vmem_limit_bytes=56 * 1024 * 1024,
        ),
    )(tok_head, emb_p, w_ih_c, w_hh_c, b_rnn)

    Tt = T - TS
    const = lambda i: (0, 0)
    out_padded = pl.pallas_call(
        functools.partial(_tail_kernel, unroll=8),
        out_shape=jax.ShapeDtypeStruct((Bp, Cp), jnp.float32),
        grid=(num_tiles,),
        in_specs=[
            pl.BlockSpec((Tt, Bt, Ep), lambda i: (0, i, 0)),
            pl.BlockSpec((Bt, Hp), lambda i: (i, 0)),
            pl.BlockSpec((Ep, Hp), const),
            pl.BlockSpec((Hp, Hp), const),
            pl.BlockSpec((1, Hp), const),
            pl.BlockSpec((Hp, Cp), const),
            pl.BlockSpec((1, Cp), const),
        ],
        out_specs=pl.BlockSpec((Bt, Cp), lambda i: (i, 0)),
        scratch_shapes=[pltpu.VMEM((Tt, Bt, Hp), jnp.bfloat16)],
        compiler_params=pltpu.CompilerParams(
            dimension_semantics=("parallel",),
        ),
    )(x_tail, h_mid, w_ih_c, w_hh_c, b_rnn, w_fc_c, b_fc_p)

    if (Bp, Cp) != (B, C):
        out_padded = out_padded[:B, :C]
    return out_padded
